# baseline (speedup 1.0000x reference)
import functools

import jax
import jax.numpy as jnp
import numpy as np
from jax import lax
from jax.experimental import pallas as pl
from jax.experimental.pallas import tpu as pltpu
from jax.experimental.pallas import tpu_sc as plsc

N_NODES = 10000
N_FEATURES = 128
N_TEMPLATES = 16
N_TNODES = 16
N_CLASSES = 8

NC = 2
NS = 16
NW = NC * NS
CHUNK = 128
NP = 10112
ROWS_PER_TILE = NP // NS
PW = NP * N_CLASSES
ACCW = PW + NP
PSLICE = PW // NS
RPAD = 5696


def _dense_body(x_ref, tf_ref, lt_ref, w_ref, b_ref, p8_ref, base_ref):
    T, n, F = N_TEMPLATES, N_TNODES, N_FEATURES
    tf2 = tf_ref[...].reshape(T * n, F)
    w = w_ref[...]
    wex = jnp.broadcast_to(w[:, None, :], (T, n, N_CLASSES)).reshape(
        T * n, N_CLASSES) * (1.0 / n)
    fbw = lax.dot_general(tf2, wex, (((0,), (0,)), ((), ())))
    f2 = jnp.sum(tf2 * tf2, axis=1, keepdims=True)
    f2w = lax.dot_general(f2, wex, (((0,), (0,)), ((), ())))
    sw = jnp.sum(w, axis=0, keepdims=True)
    struct = jnp.mean(lt_ref[...].reshape(T, n * n), axis=1, keepdims=True)
    c0 = jnp.sum(struct * w, axis=0, keepdims=True) + b_ref[...]

    x = x_ref[...]
    x2 = jnp.sum(x * x, axis=1, keepdims=True)
    p = x2 * sw + f2w - 2.0 * jnp.dot(x, fbw)
    half_p = 0.5 * p
    base_ref[...] = half_p + c0
    p8_ref[...] = jnp.concatenate(
        [half_p, jnp.zeros((NP - N_NODES, N_CLASSES), jnp.float32)], axis=0)


def _dense_part(x, tf, lt, w, b2):
    return pl.pallas_call(
        _dense_body,
        out_shape=(
            jax.ShapeDtypeStruct((NP, N_CLASSES), jnp.float32),
            jax.ShapeDtypeStruct((N_NODES, N_CLASSES), jnp.float32),
        ),
    )(x, tf, lt, w, b2)


def _make_sc_kernel(k_chunks):
    mesh = plsc.VectorSubcoreMesh(core_axis_name="c", subcore_axis_name="s")

    @functools.partial(
        pl.kernel,
        out_type=jax.ShapeDtypeStruct((NC, NP, 16), jnp.float32),
        mesh=mesh,
        scratch_types=[
            pltpu.VMEM((2, CHUNK), jnp.int32),
            pltpu.VMEM((2, CHUNK), jnp.int32),
            pltpu.VMEM((CHUNK, N_CLASSES), jnp.float32),
            pltpu.VMEM((CHUNK, N_CLASSES), jnp.float32),
            pltpu.VMEM((ACCW,), jnp.float32),
            pltpu.VMEM((RPAD,), jnp.float32),
            pltpu.VMEM((RPAD,), jnp.float32),
            pltpu.VMEM((ROWS_PER_TILE, 16), jnp.float32),
            pltpu.VMEM_SHARED((ACCW,), jnp.float32),
            pltpu.SemaphoreType.DMA,
            pltpu.SemaphoreType.DMA,
            pltpu.SemaphoreType.DMA,
            pltpu.SemaphoreType.DMA,
        ],
        compiler_params=pltpu.CompilerParams(use_tc_tiling_on_sc=False,
                                             needs_layout_passes=False),
    )
    def scatter_kernel(src_hbm, dst_hbm, p8_hbm, out_hbm,
                       idx_s, idx_d, b0, b1, acc, red, tmp, out16, parts,
                       g0, g1, i0, i1):
        c = lax.axis_index("c")
        s = lax.axis_index("s")
        wid = s * NC + c

        zero16 = jnp.zeros((16,), jnp.float32)

        def _zero(i, carry):
            acc[pl.ds(16 * i, 16)] = zero16
            return carry
        lax.fori_loop(0, ACCW // 16, _zero, None)

        lane = lax.iota(jnp.int32, 16)
        v_col = lane & 7
        v_lane8 = lane == 8
        v_mask9 = lane <= 8
        v_mult = jnp.where(lane < 8, 8, jnp.where(v_lane8, 1, 0))
        v_off = jnp.where(lane < 8, lane, jnp.where(v_lane8, PW, 0))
        ones16 = jnp.ones((16,), jnp.float32)

        bufs = (b0, b1)
        gsems = (g0, g1)
        isems = (i0, i1)

        def _prefetch(j, slot):
            pltpu.async_copy(src_hbm.at[wid, j], idx_s.at[slot], isems[slot])
            pltpu.async_copy(dst_hbm.at[wid, j], idx_d.at[slot], isems[slot])

        def _start(j, slot):
            pltpu.make_async_copy(src_hbm.at[wid, j], idx_s.at[slot],
                                  isems[slot]).wait()
            pltpu.make_async_copy(dst_hbm.at[wid, j], idx_d.at[slot],
                                  isems[slot]).wait()
            pltpu.async_copy(p8_hbm.at[idx_s.at[slot]], bufs[slot],
                             gsems[slot])

        def _process(slot):
            pltpu.make_async_copy(p8_hbm.at[idx_s.at[slot]], bufs[slot],
                                  gsems[slot]).wait()

            def _group(g, carry):
                e0 = 16 * g
                d16 = idx_d[slot, pl.ds(e0, 16)]
                for e in range(16):
                    d = d16[e]
                    row = jnp.full((16,), e0 + e, jnp.int32)
                    val = plsc.load_gather(bufs[slot], [row, v_col])
                    val = jnp.where(v_lane8, ones16, val)
                    fidx = d * v_mult + v_off
                    plsc.addupdate_scatter(acc, [fidx], val, mask=v_mask9)
                return carry
            lax.fori_loop(0, CHUNK // 16, _group, None)

        _prefetch(0, 0)
        _prefetch(1, 1)
        _start(0, 0)
        _start(1, 1)

        def _pipe(i, carry):
            j0 = 2 * i
            _process(0)

            @pl.when(j0 + 2 < k_chunks)
            def _():
                _prefetch(j0 + 2, 0)
                _start(j0 + 2, 0)
            _process(1)

            @pl.when(j0 + 3 < k_chunks)
            def _():
                _prefetch(j0 + 3, 1)
                _start(j0 + 3, 1)
            return carry
        lax.fori_loop(0, k_chunks // 2, _pipe, None)

        def _zred(i, carry):
            red[pl.ds(16 * i, 16)] = zero16
            return carry
        lax.fori_loop(0, RPAD // 16, _zred, None)

        p_lo = s * PSLICE
        d_lo = PW + s * ROWS_PER_TILE

        def _round(t, carry):
            @pl.when(s == t)
            def _():
                pltpu.sync_copy(acc, parts)
            plsc.subcore_barrier()

            pltpu.sync_copy(parts.at[pl.ds(p_lo, PSLICE)],
                            tmp.at[pl.ds(0, PSLICE)])
            pltpu.sync_copy(parts.at[pl.ds(d_lo, ROWS_PER_TILE)],
                            tmp.at[pl.ds(PSLICE, ROWS_PER_TILE)])

            def _addw(w, cc):
                plsc.addupdate(red.at[pl.ds(16 * w, 16)],
                               tmp[pl.ds(16 * w, 16)])
                return cc
            lax.fori_loop(0, RPAD // 16, _addw, None)
            plsc.subcore_barrier()
            return carry
        lax.fori_loop(0, NS, _round, None)

        v_offr = jnp.where(lane < 8, lane, jnp.where(v_lane8, PSLICE, 0))

        def _asm(n, carry):
            fidx = n * v_mult + v_offr
            out16[n, :] = plsc.load_gather(red, [fidx])
            return carry
        lax.fori_loop(0, ROWS_PER_TILE, _asm, None)
        pltpu.sync_copy(
            out16, out_hbm.at[c, pl.ds(s * ROWS_PER_TILE, ROWS_PER_TILE)])

    return scatter_kernel


def _combine_body(base_ref, ap_ref, out_ref):
    a = ap_ref[0] + ap_ref[1]
    ssum = a[:N_NODES, :N_CLASSES]
    deg = a[:N_NODES, N_CLASSES:N_CLASSES + 1]
    out_ref[...] = base_ref[...] + ssum / jnp.maximum(deg, 1.0)


def _combine(base, apart):
    return pl.pallas_call(
        _combine_body,
        out_shape=jax.ShapeDtypeStruct((N_NODES, N_CLASSES), jnp.float32),
    )(base, apart)


def kernel(x, edge_index, latent_template, templates_features, W, b):
    e = edge_index.shape[1]
    k_chunks = -(-e // (NW * CHUNK))
    k_chunks += k_chunks % 2
    per_w = k_chunks * CHUNK
    e_pad = per_w * NW

    pad = jnp.full((e_pad - e,), N_NODES, jnp.int32)
    src = jnp.concatenate([edge_index[0], pad]).reshape(NW, k_chunks, CHUNK)
    dst = jnp.concatenate([edge_index[1], pad]).reshape(NW, k_chunks, CHUNK)

    p8, base = _dense_part(x, templates_features, latent_template,
                           W, b.reshape(1, N_CLASSES))
    apart = _make_sc_kernel(k_chunks)(src, dst, p8)
    return _combine(base, apart)


# speedup vs baseline: 2.8934x; 2.8934x over previous
import functools

import jax
import jax.numpy as jnp
from jax import lax
from jax.experimental import pallas as pl
from jax.experimental.pallas import tpu as pltpu
from jax.experimental.pallas import tpu_sc as plsc

N_NODES = 10000
N_FEATURES = 128
N_TEMPLATES = 16
N_TNODES = 16
N_CLASSES = 8

NC = 2
NS = 16
NW = NC * NS
CHUNK = 128
NP = 10112
ROWS_PER_TILE = NP // NS


def _dense_body(x_ref, tf_ref, lt_ref, w_ref, b_ref, pext_ref, base_ref):
    T, n, F = N_TEMPLATES, N_TNODES, N_FEATURES
    tf2 = tf_ref[...].reshape(T * n, F)
    w = w_ref[...]
    wex = jnp.broadcast_to(w[:, None, :], (T, n, N_CLASSES)).reshape(
        T * n, N_CLASSES) * (1.0 / n)
    fbw = lax.dot_general(tf2, wex, (((0,), (0,)), ((), ())))
    f2 = jnp.sum(tf2 * tf2, axis=1, keepdims=True)
    f2w = lax.dot_general(f2, wex, (((0,), (0,)), ((), ())))
    sw = jnp.sum(w, axis=0, keepdims=True)
    struct = jnp.mean(lt_ref[...].reshape(T, n * n), axis=1, keepdims=True)
    c0 = jnp.sum(struct * w, axis=0, keepdims=True) + b_ref[...]

    x = x_ref[...]
    x2 = jnp.sum(x * x, axis=1, keepdims=True)
    p = x2 * sw + f2w - 2.0 * jnp.dot(x, fbw)
    half_p = 0.5 * p
    base_ref[...] = half_p + c0
    rows = jnp.concatenate(
        [half_p,
         jnp.ones((N_NODES, 1), jnp.float32),
         jnp.zeros((N_NODES, 16 - N_CLASSES - 1), jnp.float32)], axis=1)
    pext_ref[...] = jnp.concatenate(
        [rows, jnp.zeros((NP - N_NODES, 16), jnp.float32)], axis=0)


def _dense_part(x, tf, lt, w, b2):
    return pl.pallas_call(
        _dense_body,
        out_shape=(
            jax.ShapeDtypeStruct((NP, 16), jnp.float32),
            jax.ShapeDtypeStruct((N_NODES, N_CLASSES), jnp.float32),
        ),
    )(x, tf, lt, w, b2)


def _make_sc_kernel(k_chunks):
    mesh = plsc.VectorSubcoreMesh(core_axis_name="c", subcore_axis_name="s")

    @functools.partial(
        pl.kernel,
        out_type=jax.ShapeDtypeStruct((NC, NP, 16), jnp.float32),
        mesh=mesh,
        scratch_types=[
            pltpu.VMEM((k_chunks, CHUNK), jnp.int32),
            pltpu.VMEM((k_chunks, CHUNK), jnp.int32),
            pltpu.VMEM((8, CHUNK, 16), jnp.float32),
            pltpu.VMEM((ROWS_PER_TILE, 16), jnp.float32),
            pltpu.VMEM_SHARED((NP, 16), jnp.float32),
            pltpu.VMEM_SHARED((NP, 16), jnp.float32),
        ] + [pltpu.SemaphoreType.DMA] * 16,
        compiler_params=pltpu.CompilerParams(use_tc_tiling_on_sc=False),
    )
    def scatter_kernel(src_hbm, dst_hbm, pext_hbm, out_hbm,
                       idx_s, idx_d, bufring, stage, acc, ptab, *sems):
        c = lax.axis_index("c")
        s = lax.axis_index("s")
        wid = s * NC + c
        row0 = s * ROWS_PER_TILE

        pltpu.sync_copy(src_hbm.at[wid], idx_s)
        pltpu.sync_copy(dst_hbm.at[wid], idx_d)
        pltpu.sync_copy(pext_hbm.at[pl.ds(row0, ROWS_PER_TILE)],
                        ptab.at[pl.ds(row0, ROWS_PER_TILE)])

        def _zero_row(i, carry):
            stage[i, :] = jnp.zeros((16,), jnp.float32)
            return carry
        lax.fori_loop(0, ROWS_PER_TILE, _zero_row, None)
        pltpu.sync_copy(stage, acc.at[pl.ds(row0, ROWS_PER_TILE)])
        plsc.subcore_barrier()

        nbuf = 8
        gsems = sems[:nbuf]
        ssems = sems[nbuf:]

        def _gstart(j, slot):
            pltpu.async_copy(ptab.at[idx_s.at[j]], bufring.at[slot],
                             gsems[slot])

        def _gwait(j, slot):
            pltpu.make_async_copy(ptab.at[idx_s.at[j]], bufring.at[slot],
                                  gsems[slot]).wait()

        def _sstart(j, slot):
            pltpu.async_copy(bufring.at[slot], acc.at[idx_d.at[j]],
                             ssems[slot], add=True)

        def _swait(j, slot):
            pltpu.make_async_copy(bufring.at[slot], acc.at[idx_d.at[j]],
                                  ssems[slot]).wait()

        _gstart(0, 0)
        _gstart(1, 1)

        def _pipe(i, carry):
            j0 = nbuf * i
            for t in range(nbuf):
                j = j0 + t
                nslot = (t + 2) % nbuf

                @pl.when(j >= 6)
                def _():
                    _swait(j - 6, nslot)

                @pl.when(j + 2 < k_chunks)
                def _():
                    _gstart(j + 2, nslot)
                _gwait(j, t)
                _sstart(j, t)
            return carry
        lax.fori_loop(0, k_chunks // nbuf, _pipe, None)
        for t in range(6):
            j = k_chunks - 6 + t
            _swait(j, j % nbuf)
        plsc.subcore_barrier()

        pltpu.sync_copy(acc.at[pl.ds(row0, ROWS_PER_TILE)],
                        out_hbm.at[c, pl.ds(row0, ROWS_PER_TILE)])

    return scatter_kernel


def _combine_body(base_ref, ap_ref, out_ref):
    a = ap_ref[0] + ap_ref[1]
    ssum = a[:N_NODES, :N_CLASSES]
    deg = a[:N_NODES, N_CLASSES:N_CLASSES + 1]
    out_ref[...] = base_ref[...] + ssum / jnp.maximum(deg, 1.0)


def _combine(base, apart):
    return pl.pallas_call(
        _combine_body,
        out_shape=jax.ShapeDtypeStruct((N_NODES, N_CLASSES), jnp.float32),
    )(base, apart)


def kernel(x, edge_index, latent_template, templates_features, W, b):
    e = edge_index.shape[1]
    k_chunks = -(-e // (NW * CHUNK))
    k_chunks = -(-k_chunks // 8) * 8
    per_w = k_chunks * CHUNK
    e_pad = per_w * NW

    pad = jnp.full((e_pad - e,), N_NODES, jnp.int32)
    src = jnp.concatenate([edge_index[0], pad]).reshape(NW, k_chunks, CHUNK)
    dst = jnp.concatenate([edge_index[1], pad]).reshape(NW, k_chunks, CHUNK)

    pext, base = _dense_part(x, templates_features, latent_template,
                             W, b.reshape(1, N_CLASSES))
    apart = _make_sc_kernel(k_chunks)(src, dst, pext)
    return _combine(base, apart)
